# Initial kernel scaffold; baseline (speedup 1.0000x reference)
import jax
import jax.numpy as jnp
from jax.experimental import pallas as pl


def kernel(bilinear_coords, boxes):
    raise NotImplementedError("write your pallas kernel here")


# speedup vs baseline: 9.8630x; 9.8630x over previous
import functools

import jax
import jax.numpy as jnp
from jax.experimental import pallas as pl


def _tc_body(nq, k, x_ref, y_ref, box_ref, cent_ref, mask_ref):
    xb = x_ref[...]
    yb = y_ref[...]
    cx = box_ref[:, 0:1]
    cy = box_ref[:, 1:2]
    w = box_ref[:, 2:3]
    h = box_ref[:, 3:4]
    xmin = cx - w / 2.0
    ymin = cy - h / 2.0
    xmax = cx + w / 2.0
    ymax = cy + h / 2.0
    left = jnp.clip(xb - xmin, 0.0, 1.0)
    right = jnp.clip(xmax - xb, 0.0, 1.0)
    top = jnp.clip(yb - ymin, 0.0, 1.0)
    down = jnp.clip(ymax - yb, 0.0, 1.0)
    sx = (left + right) / 2.0
    dx = jnp.abs(left - right) / 2.0
    sy = (top + down) / 2.0
    dy = jnp.abs(top - down) / 2.0
    cxn = (sx - dx) / (sx + dx)
    cyn = (sy - dy) / (sy + dy)
    c = jnp.sqrt(cxn * cyn)
    cent_ref[...] = c
    u = jax.lax.bitcast_convert_type(c, jnp.int32)
    u = jnp.where(c == c, u, 0)

    def body(i, r):
        bit = 29 - i
        cand = r | (1 << bit)
        cnt = jnp.sum((u >= cand).astype(jnp.int32), axis=1, keepdims=True)
        return jnp.where(cnt >= k + 1, cand, r)

    thr = jax.lax.fori_loop(0, 30, body, jnp.zeros((u.shape[0], 1), jnp.int32))
    mask_ref[...] = u > thr


def kernel(bilinear_coords, boxes):
    bs, nq = bilinear_coords.shape[:2]
    k = nq // 16
    x = bilinear_coords[:, :, 0]
    y = bilinear_coords[:, :, 1]
    bb = 8
    cent, mask = pl.pallas_call(
        functools.partial(_tc_body, nq, k),
        grid=(bs // bb,),
        in_specs=[
            pl.BlockSpec((bb, nq), lambda i: (i, 0)),
            pl.BlockSpec((bb, nq), lambda i: (i, 0)),
            pl.BlockSpec((bb, 4), lambda i: (i, 0)),
        ],
        out_specs=[
            pl.BlockSpec((bb, nq), lambda i: (i, 0)),
            pl.BlockSpec((bb, nq), lambda i: (i, 0)),
        ],
        out_shape=[
            jax.ShapeDtypeStruct((bs, nq), jnp.float32),
            jax.ShapeDtypeStruct((bs, nq), jnp.bool_),
        ],
    )(x, y, boxes)
    return cent, mask
